# baseline (speedup 1.0000x reference)
import functools

import jax
import jax.numpy as jnp
from jax import lax
from jax.experimental import pallas as pl
from jax.experimental.pallas import tpu as pltpu
from jax.experimental.pallas import tpu_sc as plsc

N_NODES = 50000
K = 27
C = 32
TOTAL = N_NODES * K
G = 80
SB = 8
CH = SB * G
NW = 32
UNIT = NW * CH
NPW = -(-TOTAL // UNIT)
NPW += NPW % 2
PAD_TOTAL = NW * NPW * CH
LANES = 16
PADROW = N_NODES


def _fix_indices(idx_v):
    for i in range(CH // LANES):
        v = idx_v[pl.ds(i * LANES, LANES)]
        idx_v[pl.ds(i * LANES, LANES)] = jnp.where(v < 0, PADROW, v)


def _body(idx_hbm, data_hbm, out_hbm, i0, i1, r0, r1, is0, is1, os0, os1, gsem):
    w = lax.axis_index("s") * 2 + lax.axis_index("c")
    base = w * NPW
    last = base + NPW - 1

    def do_chunk(c_abs, ibuf, rows, isem, osem, drain_out):
        pltpu.make_async_copy(idx_hbm.at[pl.ds(0, CH)], ibuf, isem).wait()
        _fix_indices(ibuf)
        if drain_out:
            pltpu.make_async_copy(rows, out_hbm.at[pl.ds(0, CH)], osem).wait()
        cps = [
            pltpu.async_copy(
                data_hbm.at[ibuf.at[pl.ds(j * G, G)]],
                rows.at[pl.ds(j * G, G)],
                gsem,
            )
            for j in range(SB)
        ]
        for cp in cps:
            cp.wait()
        pltpu.async_copy(rows, out_hbm.at[pl.ds(c_abs * CH, CH)], osem)
        pre = jnp.minimum(c_abs + 2, last)
        pltpu.async_copy(idx_hbm.at[pl.ds(pre * CH, CH)], ibuf, isem)

    pltpu.async_copy(idx_hbm.at[pl.ds(base * CH, CH)], i0, is0)
    pltpu.async_copy(idx_hbm.at[pl.ds((base + 1) * CH, CH)], i1, is1)
    do_chunk(base, i0, r0, is0, os0, False)
    do_chunk(base + 1, i1, r1, is1, os1, False)

    def step(t, carry):
        e = base + 2 * t
        do_chunk(e, i0, r0, is0, os0, True)
        do_chunk(e + 1, i1, r1, is1, os1, True)
        return carry

    lax.fori_loop(1, NPW // 2, step, 0)

    pltpu.make_async_copy(r0, out_hbm.at[pl.ds(0, CH)], os0).wait()
    pltpu.make_async_copy(r1, out_hbm.at[pl.ds(0, CH)], os1).wait()
    pltpu.make_async_copy(idx_hbm.at[pl.ds(0, CH)], i0, is0).wait()
    pltpu.make_async_copy(idx_hbm.at[pl.ds(0, CH)], i1, is1).wait()


def kernel(data, neigh, depth):
    del depth
    data2 = jnp.concatenate([data, jnp.zeros((1, C), dtype=data.dtype)], axis=0)
    idx = neigh.astype(jnp.int32).reshape(TOTAL)
    idx = jnp.concatenate(
        [idx, jnp.full((PAD_TOTAL - TOTAL,), PADROW, jnp.int32)]
    )

    mesh = plsc.VectorSubcoreMesh(core_axis_name="c", subcore_axis_name="s")
    run = functools.partial(
        pl.kernel,
        mesh=mesh,
        out_type=jax.ShapeDtypeStruct((PAD_TOTAL, C), jnp.float32),
        scratch_types=[
            pltpu.VMEM((CH,), jnp.int32),
            pltpu.VMEM((CH,), jnp.int32),
            pltpu.VMEM((CH, C), jnp.float32),
            pltpu.VMEM((CH, C), jnp.float32),
            pltpu.SemaphoreType.DMA,
            pltpu.SemaphoreType.DMA,
            pltpu.SemaphoreType.DMA,
            pltpu.SemaphoreType.DMA,
            pltpu.SemaphoreType.DMA,
        ],
        compiler_params=pltpu.CompilerParams(use_tc_tiling_on_sc=False),
    )(_body)
    out = run(idx, data2)
    return out[:TOTAL].reshape(N_NODES, K, C)
